# baseline (speedup 1.0000x reference)
import jax
import jax.numpy as jnp
from jax.experimental import pallas as pl

_TM = 256
_NEG = -1e30


def _moe_ffn_kernel(x_ref, wr_ref, brp_ref, w1_ref, b1_ref, w2_ref,
                    b2a_ref, b2d_ref, out_ref):
    x = x_ref[...]
    logits = jnp.dot(x, wr_ref[...],
                     preferred_element_type=jnp.float32) + brp_ref[...]
    m1 = jnp.max(logits, axis=-1, keepdims=True)
    col = jax.lax.broadcasted_iota(jnp.int32, logits.shape, 1)
    fpos = jnp.min(jnp.where(logits == m1, col, logits.shape[1]),
                   axis=-1, keepdims=True)
    m2 = jnp.max(jnp.where(col == fpos, _NEG, logits), axis=-1, keepdims=True)
    w0 = jax.nn.sigmoid(m1 - m2)

    xb = x.astype(jnp.bfloat16)
    h = jnp.dot(xb, w1_ref[...], preferred_element_type=jnp.float32)
    h = jnp.maximum(h + b1_ref[...], 0.0)
    half = jax.lax.broadcasted_iota(jnp.int32, h.shape, 1) < (h.shape[1] // 2)
    h = h * jnp.where(half, w0, 1.0 - w0)
    hb = h.astype(jnp.bfloat16)
    o = jnp.dot(hb, w2_ref[...], preferred_element_type=jnp.float32)
    out_ref[...] = o + b2a_ref[...] + w0 * b2d_ref[...]


def kernel(inputs, Wr, br, W1, b1, W2, b2):
    B, S, D = inputs.shape
    T = B * S
    H2 = 2 * W1.shape[2]
    x = inputs.reshape(T, D)
    wr_pad = jnp.zeros((D, 128), Wr.dtype).at[:, :Wr.shape[1]].set(Wr)
    brp = jnp.full((1, 128), _NEG, jnp.float32).at[0, :br.shape[0]].set(br)
    w1c = jnp.concatenate([W1[0], W1[1]], axis=1).astype(jnp.bfloat16)
    b1c = jnp.concatenate([b1[0], b1[1]])[None, :]
    w2s = jnp.concatenate([W2[0], W2[1]], axis=0).astype(jnp.bfloat16)
    b2a = b2[1][None, :]
    b2d = (b2[0] - b2[1])[None, :]

    out = pl.pallas_call(
        _moe_ffn_kernel,
        grid=(T // _TM,),
        in_specs=[
            pl.BlockSpec((_TM, D), lambda i: (i, 0)),
            pl.BlockSpec((D, 128), lambda i: (0, 0)),
            pl.BlockSpec((1, 128), lambda i: (0, 0)),
            pl.BlockSpec((D, H2), lambda i: (0, 0)),
            pl.BlockSpec((1, H2), lambda i: (0, 0)),
            pl.BlockSpec((H2, D), lambda i: (0, 0)),
            pl.BlockSpec((1, D), lambda i: (0, 0)),
            pl.BlockSpec((1, D), lambda i: (0, 0)),
        ],
        out_specs=pl.BlockSpec((_TM, D), lambda i: (i, 0)),
        out_shape=jax.ShapeDtypeStruct((T, D), jnp.float32),
    )(x, wr_pad, brp, w1c, b1c, w2s, b2a, b2d)
    return out.reshape(B, S, D)


# speedup vs baseline: 1.0206x; 1.0206x over previous
import jax
import jax.numpy as jnp
from jax.experimental import pallas as pl

_TM = 512
_NEG = -1e30


def _moe_ffn_kernel(x_ref, wr_ref, brp_ref, w1_ref, b1_ref, w20_ref, w21_ref,
                    b2a_ref, b2d_ref, out_ref):
    x = x_ref[...]
    logits = jnp.dot(x, wr_ref[...],
                     preferred_element_type=jnp.float32) + brp_ref[...]
    m1 = jnp.max(logits, axis=-1, keepdims=True)
    col = jax.lax.broadcasted_iota(jnp.int32, logits.shape, 1)
    fpos = jnp.min(jnp.where(logits == m1, col, logits.shape[1]),
                   axis=-1, keepdims=True)
    m2 = jnp.max(jnp.where(col == fpos, _NEG, logits), axis=-1, keepdims=True)
    w0 = jax.nn.sigmoid(m1 - m2)

    xb = x.astype(jnp.bfloat16)
    h32 = jnp.dot(xb, w1_ref[...], preferred_element_type=jnp.float32)
    h = jnp.maximum(h32 + b1_ref[...], 0.0).astype(jnp.bfloat16)
    H = h.shape[1] // 2
    o0 = jnp.dot(h[:, :H], w20_ref[...], preferred_element_type=jnp.float32)
    o1 = jnp.dot(h[:, H:], w21_ref[...], preferred_element_type=jnp.float32)
    out_ref[...] = o1 + w0 * (o0 - o1) + b2a_ref[...] + w0 * b2d_ref[...]


def kernel(inputs, Wr, br, W1, b1, W2, b2):
    B, S, D = inputs.shape
    T = B * S
    Hid = W1.shape[2]
    x = inputs.reshape(T, D)
    wr_pad = jnp.zeros((D, 128), Wr.dtype).at[:, :Wr.shape[1]].set(Wr)
    brp = jnp.full((1, 128), _NEG, jnp.float32).at[0, :br.shape[0]].set(br)
    w1c = jnp.concatenate([W1[0], W1[1]], axis=1).astype(jnp.bfloat16)
    b1c = jnp.concatenate([b1[0], b1[1]])[None, :]
    w20 = W2[0].astype(jnp.bfloat16)
    w21 = W2[1].astype(jnp.bfloat16)
    b2a = b2[1][None, :]
    b2d = (b2[0] - b2[1])[None, :]

    out = pl.pallas_call(
        _moe_ffn_kernel,
        grid=(T // _TM,),
        in_specs=[
            pl.BlockSpec((_TM, D), lambda i: (i, 0)),
            pl.BlockSpec((D, 128), lambda i: (0, 0)),
            pl.BlockSpec((1, 128), lambda i: (0, 0)),
            pl.BlockSpec((D, 2 * Hid), lambda i: (0, 0)),
            pl.BlockSpec((1, 2 * Hid), lambda i: (0, 0)),
            pl.BlockSpec((Hid, D), lambda i: (0, 0)),
            pl.BlockSpec((Hid, D), lambda i: (0, 0)),
            pl.BlockSpec((1, D), lambda i: (0, 0)),
            pl.BlockSpec((1, D), lambda i: (0, 0)),
        ],
        out_specs=pl.BlockSpec((_TM, D), lambda i: (i, 0)),
        out_shape=jax.ShapeDtypeStruct((T, D), jnp.float32),
    )(x, wr_pad, brp, w1c, b1c, w20, w21, b2a, b2d)
    return out.reshape(B, S, D)


# speedup vs baseline: 1.7205x; 1.6858x over previous
import jax
import jax.numpy as jnp
from jax.experimental import pallas as pl

_TM = 256
_NEG = -1e30


def _router_w0(x, wr, brp):
    logits = jnp.dot(x, wr, preferred_element_type=jnp.float32) + brp
    m1 = jnp.max(logits, axis=-1, keepdims=True)
    col = jax.lax.broadcasted_iota(jnp.int32, logits.shape, 1)
    fpos = jnp.min(jnp.where(logits == m1, col, logits.shape[1]),
                   axis=-1, keepdims=True)
    m2 = jnp.max(jnp.where(col == fpos, _NEG, logits), axis=-1, keepdims=True)
    return jax.nn.sigmoid(m1 - m2)


def _expert0_kernel(x_ref, wr_ref, brp_ref, w1_ref, b1_ref, w2_ref, b2_ref,
                    out_ref):
    x = x_ref[...]
    w0 = _router_w0(x, wr_ref[...], brp_ref[...])
    h = jnp.maximum(jnp.dot(x, w1_ref[0],
                            preferred_element_type=jnp.float32)
                    + b1_ref[0], 0.0)
    o = jnp.dot(h, w2_ref[0], preferred_element_type=jnp.float32)
    out_ref[...] = w0 * (o + b2_ref[0])


def _expert1_kernel(x_ref, wr_ref, brp_ref, w1_ref, b1_ref, w2_ref, b2_ref,
                    prev_ref, out_ref):
    x = x_ref[...]
    w0 = _router_w0(x, wr_ref[...], brp_ref[...])
    h = jnp.maximum(jnp.dot(x, w1_ref[0],
                            preferred_element_type=jnp.float32)
                    + b1_ref[0], 0.0)
    o = jnp.dot(h, w2_ref[0], preferred_element_type=jnp.float32)
    out_ref[...] = prev_ref[...] + (1.0 - w0) * (o + b2_ref[0])


def kernel(inputs, Wr, br, W1, b1, W2, b2):
    B, S, D = inputs.shape
    T = B * S
    Hid = W1.shape[2]
    x = inputs.reshape(T, D)
    wr_pad = jnp.zeros((D, 128), Wr.dtype).at[:, :Wr.shape[1]].set(Wr)
    brp = jnp.full((1, 128), _NEG, jnp.float32).at[0, :br.shape[0]].set(br)
    b1r = b1.reshape(b1.shape[0], 1, Hid)
    b2r = b2.reshape(b2.shape[0], 1, D)

    def specs(e, with_prev):
        s = [
            pl.BlockSpec((_TM, D), lambda i: (i, 0)),
            pl.BlockSpec((D, 128), lambda i: (0, 0)),
            pl.BlockSpec((1, 128), lambda i: (0, 0)),
            pl.BlockSpec((1, D, Hid), lambda i: (e, 0, 0)),
            pl.BlockSpec((1, 1, Hid), lambda i: (e, 0, 0)),
            pl.BlockSpec((1, Hid, D), lambda i: (e, 0, 0)),
            pl.BlockSpec((1, 1, D), lambda i: (e, 0, 0)),
        ]
        if with_prev:
            s.append(pl.BlockSpec((_TM, D), lambda i: (i, 0)))
        return s

    grid = (T // _TM,)
    out_sd = jax.ShapeDtypeStruct((T, D), jnp.float32)
    part = pl.pallas_call(
        _expert0_kernel, grid=grid,
        in_specs=specs(0, False),
        out_specs=pl.BlockSpec((_TM, D), lambda i: (i, 0)),
        out_shape=out_sd,
    )(x, wr_pad, brp, W1, b1r, W2, b2r)
    out = pl.pallas_call(
        _expert1_kernel, grid=grid,
        in_specs=specs(1, True),
        out_specs=pl.BlockSpec((_TM, D), lambda i: (i, 0)),
        out_shape=out_sd,
        input_output_aliases={7: 0},
    )(x, wr_pad, brp, W1, b1r, W2, b2r, part)
    return out.reshape(B, S, D)


# speedup vs baseline: 1.7822x; 1.0358x over previous
import jax
import jax.numpy as jnp
from jax.experimental import pallas as pl

_TM = 512
_NEG = -1e30


def _router_w0(x, wr, br):
    logits = jnp.dot(x, wr, preferred_element_type=jnp.float32) + br
    m1 = jnp.max(logits, axis=-1, keepdims=True)
    col = jax.lax.broadcasted_iota(jnp.int32, logits.shape, 1)
    fpos = jnp.min(jnp.where(logits == m1, col, logits.shape[1]),
                   axis=-1, keepdims=True)
    m2 = jnp.max(jnp.where(col == fpos, _NEG, logits), axis=-1, keepdims=True)
    return jax.nn.sigmoid(m1 - m2)


def _expert0_kernel(x_ref, wr_ref, br_ref, w1_ref, b1_ref, w2_ref, b2_ref,
                    out_ref):
    x = x_ref[...]
    w0 = _router_w0(x, wr_ref[...], br_ref[...])
    h = jnp.maximum(jnp.dot(x, w1_ref[0],
                            preferred_element_type=jnp.float32)
                    + b1_ref[0], 0.0)
    o = jnp.dot(h, w2_ref[0], preferred_element_type=jnp.float32)
    out_ref[...] = w0 * (o + b2_ref[0])


def _expert1_kernel(x_ref, wr_ref, br_ref, w1_ref, b1_ref, w2_ref, b2_ref,
                    prev_ref, out_ref):
    x = x_ref[...]
    w0 = _router_w0(x, wr_ref[...], br_ref[...])
    h = jnp.maximum(jnp.dot(x, w1_ref[0],
                            preferred_element_type=jnp.float32)
                    + b1_ref[0], 0.0)
    o = jnp.dot(h, w2_ref[0], preferred_element_type=jnp.float32)
    out_ref[...] = prev_ref[...] + (1.0 - w0) * (o + b2_ref[0])


def kernel(inputs, Wr, br, W1, b1, W2, b2):
    B, S, D = inputs.shape
    T = B * S
    E = Wr.shape[1]
    Hid = W1.shape[2]
    x = inputs.reshape(T, D)
    brr = br.reshape(1, E)
    b1r = b1.reshape(b1.shape[0], 1, Hid)
    b2r = b2.reshape(b2.shape[0], 1, D)

    def specs(e, with_prev):
        s = [
            pl.BlockSpec((_TM, D), lambda i: (i, 0)),
            pl.BlockSpec((D, E), lambda i: (0, 0)),
            pl.BlockSpec((1, E), lambda i: (0, 0)),
            pl.BlockSpec((1, D, Hid), lambda i: (e, 0, 0)),
            pl.BlockSpec((1, 1, Hid), lambda i: (e, 0, 0)),
            pl.BlockSpec((1, Hid, D), lambda i: (e, 0, 0)),
            pl.BlockSpec((1, 1, D), lambda i: (e, 0, 0)),
        ]
        if with_prev:
            s.append(pl.BlockSpec((_TM, D), lambda i: (i, 0)))
        return s

    grid = (T // _TM,)
    out_sd = jax.ShapeDtypeStruct((T, D), jnp.float32)
    part = pl.pallas_call(
        _expert0_kernel, grid=grid,
        in_specs=specs(0, False),
        out_specs=pl.BlockSpec((_TM, D), lambda i: (i, 0)),
        out_shape=out_sd,
    )(x, Wr, brr, W1, b1r, W2, b2r)
    out = pl.pallas_call(
        _expert1_kernel, grid=grid,
        in_specs=specs(1, True),
        out_specs=pl.BlockSpec((_TM, D), lambda i: (i, 0)),
        out_shape=out_sd,
        input_output_aliases={7: 0},
    )(x, Wr, brr, W1, b1r, W2, b2r, part)
    return out.reshape(B, S, D)


# speedup vs baseline: 1.8181x; 1.0202x over previous
import functools

import jax
import jax.numpy as jnp
from jax.experimental import pallas as pl
from jax.experimental.pallas import tpu as pltpu

_TM = 512
_NEG = -1e30


def _router_w0(x, wr, br):
    logits = jnp.dot(x, wr, preferred_element_type=jnp.float32) + br
    m1 = jnp.max(logits, axis=-1, keepdims=True)
    col = jax.lax.broadcasted_iota(jnp.int32, logits.shape, 1)
    fpos = jnp.min(jnp.where(logits == m1, col, logits.shape[1]),
                   axis=-1, keepdims=True)
    m2 = jnp.max(jnp.where(col == fpos, _NEG, logits), axis=-1, keepdims=True)
    return jax.nn.sigmoid(m1 - m2)


def _expert_kernel(x_ref, wr_ref, br_ref, w1_hbm, b1_ref, w2_hbm, b2_ref,
                   *rest, expert, first_expert):
    if first_expert:
        out_ref, w1_vmem, w2_vmem, sem1, sem2 = rest
    else:
        prev_ref, out_ref, w1_vmem, w2_vmem, sem1, sem2 = rest
    first = pl.program_id(0) == 0

    cp1 = pltpu.make_async_copy(w1_hbm.at[expert], w1_vmem, sem1)
    cp2 = pltpu.make_async_copy(w2_hbm.at[expert], w2_vmem, sem2)

    @pl.when(first)
    def _start():
        cp1.start()
        cp2.start()
        cp1.wait()

    x = x_ref[...]
    w0 = _router_w0(x, wr_ref[...], br_ref[...])
    h = jnp.maximum(jnp.dot(x, w1_vmem[...],
                            preferred_element_type=jnp.float32)
                    + b1_ref[0], 0.0)

    @pl.when(first)
    def _wait_w2():
        cp2.wait()

    o = jnp.dot(h, w2_vmem[...], preferred_element_type=jnp.float32)
    if first_expert:
        out_ref[...] = w0 * (o + b2_ref[0])
    else:
        out_ref[...] = rest[0][...] + (1.0 - w0) * (o + b2_ref[0])


def kernel(inputs, Wr, br, W1, b1, W2, b2):
    B, S, D = inputs.shape
    T = B * S
    E = Wr.shape[1]
    Hid = W1.shape[2]
    x = inputs.reshape(T, D)
    brr = br.reshape(1, E)
    b1r = b1.reshape(b1.shape[0], 1, Hid)
    b2r = b2.reshape(b2.shape[0], 1, D)

    def specs(e, with_prev):
        s = [
            pl.BlockSpec((_TM, D), lambda i: (i, 0)),
            pl.BlockSpec((D, E), lambda i: (0, 0)),
            pl.BlockSpec((1, E), lambda i: (0, 0)),
            pl.BlockSpec(memory_space=pl.ANY),
            pl.BlockSpec((1, 1, Hid), lambda i: (e, 0, 0)),
            pl.BlockSpec(memory_space=pl.ANY),
            pl.BlockSpec((1, 1, D), lambda i: (e, 0, 0)),
        ]
        if with_prev:
            s.append(pl.BlockSpec((_TM, D), lambda i: (i, 0)))
        return s

    scratch = [
        pltpu.VMEM((D, Hid), jnp.float32),
        pltpu.VMEM((Hid, D), jnp.float32),
        pltpu.SemaphoreType.DMA,
        pltpu.SemaphoreType.DMA,
    ]
    grid = (T // _TM,)
    out_sd = jax.ShapeDtypeStruct((T, D), jnp.float32)
    part = pl.pallas_call(
        functools.partial(_expert_kernel, expert=0, first_expert=True),
        grid=grid,
        in_specs=specs(0, False),
        out_specs=pl.BlockSpec((_TM, D), lambda i: (i, 0)),
        out_shape=out_sd,
        scratch_shapes=scratch,
    )(x, Wr, brr, W1, b1r, W2, b2r)
    out = pl.pallas_call(
        functools.partial(_expert_kernel, expert=1, first_expert=False),
        grid=grid,
        in_specs=specs(1, True),
        out_specs=pl.BlockSpec((_TM, D), lambda i: (i, 0)),
        out_shape=out_sd,
        scratch_shapes=scratch,
        input_output_aliases={7: 0},
    )(x, Wr, brr, W1, b1r, W2, b2r, part)
    return out.reshape(B, S, D)
